# Initial kernel scaffold; baseline (speedup 1.0000x reference)
import jax
import jax.numpy as jnp
from jax.experimental import pallas as pl


def kernel(x, ts_hour, ts_minute, ts_second, ts_dayofweek, W_hour, W_minute, W_second, W_dayofweek):
    raise NotImplementedError("write your pallas kernel here")


# speedup vs baseline: 4.2006x; 4.2006x over previous
import functools

import jax
import jax.numpy as jnp
from jax import lax
from jax.experimental import pallas as pl
from jax.experimental.pallas import tpu as pltpu
from jax.experimental.pallas import tpu_sc as plsc

B, L, D = 4096, 200, 64
N = B * L
M = N // 2
NC, NS = 2, 16
NW = NC * NS
ROWS_PER_W = M // NW
GW = 128
HKW = 256
SUBW = HKW // GW
LANES = 16


def _sc_encode(x2, ime, imo, ise, iso, ihe, iho, ide, ido,
               t1l, t1r, t2l, t2r):
    mesh = plsc.VectorSubcoreMesh(core_axis_name="c", subcore_axis_name="s")

    idx_t = pltpu.VMEM((HKW,), jnp.int32)

    @functools.partial(
        pl.kernel,
        out_type=jax.ShapeDtypeStruct((M, 128), jnp.float32),
        mesh=mesh,
        scratch_types=[
            pltpu.VMEM((HKW, 128), jnp.float32),
            pltpu.VMEM((HKW, 128), jnp.float32),
        ] + [idx_t] * 16 + [
            idx_t, idx_t, idx_t, idx_t,
            idx_t, idx_t, idx_t, idx_t,
            pltpu.SemaphoreType.DMA,
            pltpu.SemaphoreType.DMA,
            pltpu.SemaphoreType.DMA,
            pltpu.SemaphoreType.DMA,
            pltpu.SemaphoreType.DMA,
            pltpu.SemaphoreType.DMA,
        ],
    )
    def k(x_hbm, ime_hbm, imo_hbm, ise_hbm, iso_hbm,
          ihe_hbm, iho_hbm, ide_hbm, ido_hbm,
          t1l_hbm, t1r_hbm, t2l_hbm, t2r_hbm, o_hbm,
          acc0, acc1,
          ime0, imo0, ise0, iso0, ihe0, iho0, ide0, ido0,
          ime1, imo1, ise1, iso1, ihe1, iho1, ide1, ido1,
          c1e0, c1o0, c2e0, c2o0, c1e1, c1o1, c2e1, c2o1,
          sl0, sl1, sa0, sa1, so0, so1):
        wid = lax.axis_index("s") * NC + lax.axis_index("c")
        base = wid * ROWS_PER_W
        accs = (acc0, acc1)
        raw = (
            ((ime0, ime_hbm), (imo0, imo_hbm), (ise0, ise_hbm),
             (iso0, iso_hbm), (ihe0, ihe_hbm), (iho0, iho_hbm),
             (ide0, ide_hbm), (ido0, ido_hbm)),
            ((ime1, ime_hbm), (imo1, imo_hbm), (ise1, ise_hbm),
             (iso1, iso_hbm), (ihe1, ihe_hbm), (iho1, iho_hbm),
             (ide1, ide_hbm), (ido1, ido_hbm)),
        )
        combos = (
            ((c1e0, ime0, ise0, 61, t1l_hbm), (c1o0, imo0, iso0, 61, t1r_hbm),
             (c2e0, ihe0, ide0, 8, t2l_hbm), (c2o0, iho0, ido0, 8, t2r_hbm)),
            ((c1e1, ime1, ise1, 61, t1l_hbm), (c1o1, imo1, iso1, 61, t1r_hbm),
             (c2e1, ihe1, ide1, 8, t2l_hbm), (c2o1, iho1, ido1, 8, t2r_hbm)),
        )
        sls, sas, sos = (sl0, sl1), (sa0, sa1), (so0, so1)

        def fire_loads(b, off):
            pltpu.async_copy(x_hbm.at[pl.ds(off, HKW)], accs[b], sls[b])
            for buf, src in raw[b]:
                pltpu.async_copy(src.at[pl.ds(off, HKW)], buf, sls[b])

        def wait_loads(b):
            pltpu.make_async_copy(x_hbm.at[pl.ds(0, HKW)], accs[b],
                                  sls[b]).wait()
            for buf, src in raw[b]:
                pltpu.make_async_copy(src.at[pl.ds(0, HKW)], buf,
                                      sls[b]).wait()

        def combine_indices(b):
            for dst, a, c, mult, _ in combos[b]:
                for j in range(0, HKW, LANES):
                    s = pl.ds(j, LANES)
                    dst.at[s][...] = a.at[s][...] * mult + c.at[s][...]

        def fire_adds(b):
            for dst, _, _, _, tbl in combos[b]:
                for j in range(SUBW):
                    w = pl.ds(j * GW, GW)
                    pltpu.async_copy(tbl.at[dst.at[w]], accs[b].at[w],
                                     sas[b], add=True)

        def wait_adds(b):
            for dst, _, _, _, tbl in combos[b]:
                for j in range(SUBW):
                    w = pl.ds(j * GW, GW)
                    pltpu.make_async_copy(tbl.at[dst.at[w]], accs[b].at[w],
                                          sas[b]).wait()

        def fire_store(b, off):
            pltpu.async_copy(accs[b], o_hbm.at[pl.ds(off, HKW)], sos[b])

        def wait_store(b):
            pltpu.make_async_copy(accs[b], o_hbm.at[pl.ds(0, HKW)],
                                  sos[b]).wait()

        fire_loads(0, base)
        fire_loads(1, base + HKW)

        @pl.loop(0, ROWS_PER_W, step=2 * HKW)
        def _(r):
            off_a = base + r
            off_b = off_a + HKW
            off_c = off_b + HKW
            off_d = off_c + HKW

            wait_loads(0)
            combine_indices(0)
            fire_adds(0)

            wait_loads(1)
            combine_indices(1)

            wait_adds(0)
            fire_store(0, off_a)
            fire_adds(1)

            wait_store(0)

            @pl.when(r + 2 * HKW < ROWS_PER_W)
            def _():
                fire_loads(0, off_c)

            wait_adds(1)
            fire_store(1, off_b)
            wait_store(1)

            @pl.when(r + 3 * HKW < ROWS_PER_W)
            def _():
                fire_loads(1, off_d)

    return k(x2, ime, imo, ise, iso, ihe, iho, ide, ido,
             t1l, t1r, t2l, t2r)


@jax.jit
def kernel(x, ts_hour, ts_minute, ts_second, ts_dayofweek,
           W_hour, W_minute, W_second, W_dayofweek):
    t1 = (W_minute[:, None, :] + W_second[None, :, :]).reshape(61 * 61, D)
    t2 = (W_hour[:, None, :] + W_dayofweek[None, :, :]).reshape(25 * 8, D)
    z1 = jnp.zeros_like(t1)
    z2 = jnp.zeros_like(t2)
    t1l = jnp.concatenate([t1, z1], axis=1)
    t1r = jnp.concatenate([z1, t1], axis=1)
    t2l = jnp.concatenate([t2, z2], axis=1)
    t2r = jnp.concatenate([z2, t2], axis=1)

    x2 = x.reshape(M, 128)

    def split(ts):
        t = ts.reshape(M, 2)
        return t[:, 0], t[:, 1]

    ime, imo = split(ts_minute)
    ise, iso = split(ts_second)
    ihe, iho = split(ts_hour)
    ide, ido = split(ts_dayofweek)

    out = _sc_encode(x2, ime, imo, ise, iso, ihe, iho, ide, ido,
                     t1l, t1r, t2l, t2r)
    return out.reshape(B, L, D)


# speedup vs baseline: 6.5537x; 1.5602x over previous
import functools

import jax
import jax.numpy as jnp
from jax import lax
from jax.experimental import pallas as pl
from jax.experimental.pallas import tpu as pltpu
from jax.experimental.pallas import tpu_sc as plsc

B, L, D = 4096, 200, 64
N = B * L
NC, NS = 2, 16
NW = NC * NS
ROWS_PER_W = N // NW
GW = 128
SUB = 4
KW = GW * SUB
LANES = 16


def _sc_encode(xf, im, isec, ih, idow, t1, t2):
    mesh = plsc.VectorSubcoreMesh(core_axis_name="c", subcore_axis_name="s")

    @functools.partial(
        pl.kernel,
        out_type=jax.ShapeDtypeStruct((N, D), jnp.float32),
        mesh=mesh,
        scratch_types=[
            pltpu.VMEM((KW, D), jnp.float32),
            pltpu.VMEM((KW, D), jnp.float32),
            pltpu.VMEM((KW,), jnp.int32),
            pltpu.VMEM((KW,), jnp.int32),
            pltpu.VMEM((KW,), jnp.int32),
            pltpu.VMEM((KW,), jnp.int32),
            pltpu.VMEM((KW,), jnp.int32),
            pltpu.VMEM((KW,), jnp.int32),
            pltpu.VMEM((KW,), jnp.int32),
            pltpu.VMEM((KW,), jnp.int32),
            pltpu.VMEM((KW,), jnp.int32),
            pltpu.VMEM((KW,), jnp.int32),
            pltpu.VMEM((KW,), jnp.int32),
            pltpu.VMEM((KW,), jnp.int32),
            pltpu.SemaphoreType.DMA,
            pltpu.SemaphoreType.DMA,
            pltpu.SemaphoreType.DMA,
            pltpu.SemaphoreType.DMA,
            pltpu.SemaphoreType.DMA,
            pltpu.SemaphoreType.DMA,
        ],
        compiler_params=pltpu.CompilerParams(use_tc_tiling_on_sc=False),
    )
    def k(x_hbm, im_hbm, is_hbm, ih_hbm, id_hbm, t1_hbm, t2_hbm, o_hbm,
          acc0, acc1, im0, is0, ih0, id0, im1, is1, ih1, id1,
          c1v0, c2v0, c1v1, c2v1, sl0, sl1, sa0, sa1, so0, so1):
        wid = lax.axis_index("s") * NC + lax.axis_index("c")
        base = wid * ROWS_PER_W
        accs = (acc0, acc1)
        imvs, isvs = (im0, im1), (is0, is1)
        ihvs, idvs = (ih0, ih1), (id0, id1)
        c1vs, c2vs = (c1v0, c1v1), (c2v0, c2v1)
        sls, sas, sos = (sl0, sl1), (sa0, sa1), (so0, so1)

        def fire_loads(b, off):
            pltpu.async_copy(x_hbm.at[pl.ds(off, KW)], accs[b], sls[b])
            pltpu.async_copy(im_hbm.at[pl.ds(off, KW)], imvs[b], sls[b])
            pltpu.async_copy(is_hbm.at[pl.ds(off, KW)], isvs[b], sls[b])
            pltpu.async_copy(ih_hbm.at[pl.ds(off, KW)], ihvs[b], sls[b])
            pltpu.async_copy(id_hbm.at[pl.ds(off, KW)], idvs[b], sls[b])

        def wait_loads(b):
            pltpu.make_async_copy(x_hbm.at[pl.ds(0, KW)], accs[b], sls[b]).wait()
            pltpu.make_async_copy(im_hbm.at[pl.ds(0, KW)], imvs[b], sls[b]).wait()
            pltpu.make_async_copy(is_hbm.at[pl.ds(0, KW)], isvs[b], sls[b]).wait()
            pltpu.make_async_copy(ih_hbm.at[pl.ds(0, KW)], ihvs[b], sls[b]).wait()
            pltpu.make_async_copy(id_hbm.at[pl.ds(0, KW)], idvs[b], sls[b]).wait()

        def combine_indices(b):
            for j in range(0, KW, LANES):
                s = pl.ds(j, LANES)
                c1vs[b].at[s][...] = imvs[b].at[s][...] * 61 + isvs[b].at[s][...]
                c2vs[b].at[s][...] = ihvs[b].at[s][...] * 8 + idvs[b].at[s][...]

        def fire_adds(b):
            for j in range(SUB):
                w = pl.ds(j * GW, GW)
                pltpu.async_copy(t1_hbm.at[c1vs[b].at[w]], accs[b].at[w],
                                 sas[b], add=True)
                pltpu.async_copy(t2_hbm.at[c2vs[b].at[w]], accs[b].at[w],
                                 sas[b], add=True)

        def wait_adds(b):
            for j in range(SUB):
                w = pl.ds(j * GW, GW)
                pltpu.make_async_copy(t1_hbm.at[c1vs[b].at[w]], accs[b].at[w],
                                      sas[b]).wait()
                pltpu.make_async_copy(t2_hbm.at[c2vs[b].at[w]], accs[b].at[w],
                                      sas[b]).wait()

        def fire_store(b, off):
            pltpu.async_copy(accs[b], o_hbm.at[pl.ds(off, KW)], sos[b])

        def wait_store(b):
            pltpu.make_async_copy(accs[b], o_hbm.at[pl.ds(0, KW)], sos[b]).wait()

        fire_loads(0, base)
        fire_loads(1, base + KW)

        @pl.loop(0, ROWS_PER_W, step=2 * KW)
        def _(r):
            off_a = base + r
            off_b = off_a + KW
            off_c = off_b + KW
            off_d = off_c + KW

            wait_loads(0)
            combine_indices(0)
            fire_adds(0)

            wait_loads(1)
            combine_indices(1)
            fire_adds(1)

            wait_adds(0)
            fire_store(0, off_a)

            wait_store(0)

            @pl.when(r + 2 * KW < ROWS_PER_W)
            def _():
                fire_loads(0, off_c)

            wait_adds(1)
            fire_store(1, off_b)
            wait_store(1)

            @pl.when(r + 3 * KW < ROWS_PER_W)
            def _():
                fire_loads(1, off_d)

    return k(xf, im, isec, ih, idow, t1, t2)


@jax.jit
def kernel(x, ts_hour, ts_minute, ts_second, ts_dayofweek,
           W_hour, W_minute, W_second, W_dayofweek):
    t1 = (W_minute[:, None, :] + W_second[None, :, :]).reshape(61 * 61, D)
    t2 = (W_hour[:, None, :] + W_dayofweek[None, :, :]).reshape(25 * 8, D)
    xf = x.reshape(N, D)
    im = ts_minute.reshape(N)
    isec = ts_second.reshape(N)
    ih = ts_hour.reshape(N)
    idow = ts_dayofweek.reshape(N)
    out = _sc_encode(xf, im, isec, ih, idow, t1, t2)
    return out.reshape(B, L, D)
